# baseline (speedup 1.0000x reference)
import jax
import jax.numpy as jnp
from jax.experimental import pallas as pl

K = 20
EPS = 1e-5


def _knn(x, k):
    inner = -2.0 * jnp.matmul(jnp.swapaxes(x, 2, 1), x)
    xx = jnp.sum(x ** 2, axis=1, keepdims=True)
    pairwise_distance = -xx - inner - jnp.swapaxes(xx, 2, 1)
    idx = jax.lax.top_k(pairwise_distance, k)[1]
    return idx


def _get_graph_feature(x, k):
    B, C, N = x.shape
    idx = _knn(x, k)
    xt = jnp.swapaxes(x, 2, 1)
    b_idx = jnp.arange(B)[:, None, None]
    feature = xt[b_idx, idx]
    xc = jnp.broadcast_to(xt[:, :, None, :], (B, N, k, C))
    feature = jnp.concatenate((feature - xc, xc), axis=3)
    return jnp.transpose(feature, (0, 3, 1, 2))


def _bn(y, g, b):
    mean = jnp.mean(y, axis=(0, 2, 3), keepdims=True)
    var = jnp.var(y, axis=(0, 2, 3), keepdims=True)
    yhat = (y - mean) / jnp.sqrt(var + EPS)
    return yhat * g[None, :, None, None] + b[None, :, None, None]


def _block(x, W, g, b, k):
    feat = _get_graph_feature(x, k)
    y = jnp.einsum('oc,bcnk->bonk', W, feat)
    y = _bn(y, g, b)
    y = jnp.where(y >= 0, y, 0.2 * y)
    return jnp.max(y, axis=-1)


def _copy_kernel(x_ref, o_ref):
    o_ref[...] = x_ref[...]


def kernel(x, W1, g1, b1, W2, g2, b2, W3, g3, b3, W4, g4, b4):
    xt = jnp.swapaxes(x, 1, 2)
    x1 = _block(xt, W1, g1, b1, K)
    x2 = _block(x1, W2, g2, b2, K)
    x3 = _block(x2, W3, g3, b3, K)
    x4 = _block(x3, W4, g4, b4, K)
    out = jnp.swapaxes(x4, 1, 2)
    return pl.pallas_call(
        _copy_kernel,
        out_shape=jax.ShapeDtypeStruct(out.shape, out.dtype),
    )(out)


# speedup vs baseline: 7.5381x; 7.5381x over previous
import functools

import jax
import jax.numpy as jnp
from jax import lax
from jax.experimental import pallas as pl
from jax.experimental.pallas import tpu as pltpu
from jax.experimental.pallas import tpu_sc as plsc

K = 20
EPS = 1e-5
_NEG = -1e30
_NW = 32
_CH = 8
_XP = 128


def _tc_knn(xt, xs):
    B, N, Cin = xt.shape
    R = 256

    def body(xt_tile_ref, xs_full_ref, idx_ref):
        b = pl.program_id(0)
        xt_t = xt_tile_ref[0]
        xsf = xs_full_ref[0]
        dot = lax.dot_general(xt_t, xsf, (((1,), (0,)), ((), ())),
                              preferred_element_type=jnp.float32)
        xxl = jnp.sum(xsf * xsf, axis=0, keepdims=True)
        d = 2.0 * dot - xxl
        lane = lax.broadcasted_iota(jnp.int32, (R, N), 1)
        cols = []
        for _ in range(K):
            m = jnp.max(d, axis=1, keepdims=True)
            cand = jnp.where(d == m, lane, N)
            am = jnp.min(cand, axis=1, keepdims=True)
            cols.append(am)
            d = jnp.where(cand == am, _NEG, d)
        idx_ref[0] = jnp.concatenate(cols, axis=1) + b * N

    return pl.pallas_call(
        body,
        grid=(B, N // R),
        in_specs=[
            pl.BlockSpec((1, R, Cin), lambda b, i: (b, i, 0)),
            pl.BlockSpec((1, Cin, N), lambda b, i: (b, 0, 0)),
        ],
        out_specs=pl.BlockSpec((1, R, K), lambda b, i: (b, i, 0)),
        out_shape=jax.ShapeDtypeStruct((B, N, K), jnp.int32),
    )(xt, xs)


def _sc_feat(xp2, xsh, idxf, C, FW):
    BN = xp2.shape[0]
    PW = BN // _NW
    NCH = PW // _CH
    HK = _CH * K // 2
    mesh = plsc.VectorSubcoreMesh(core_axis_name="c", subcore_axis_name="s")

    @functools.partial(
        pl.kernel, mesh=mesh,
        out_type=jax.ShapeDtypeStruct((K * BN, FW), jnp.float32),
        scratch_types=[
            pltpu.VMEM((PW * K,), jnp.int32),
            pltpu.VMEM((_CH * K, _XP), jnp.float32),
            pltpu.VMEM((_CH, _XP), jnp.float32),
            pltpu.VMEM((_CH, _XP), jnp.float32),
            pltpu.VMEM((K * _CH, FW), jnp.float32),
            pltpu.SemaphoreType.DMA,
        ],
    )
    def sc_kernel(xp_hbm, xsh_hbm, idx_hbm, feat_hbm,
                  idx_v, rows_v, xc_v, xsh_v, feat_v, sem):
        wid = lax.axis_index("s") * 2 + lax.axis_index("c")
        pbase = wid * PW
        pltpu.sync_copy(idx_hbm.at[pl.ds(pbase * K, PW * K)], idx_v)
        if C < 16:
            def zrow(r, carry):
                z = jnp.zeros((16,), jnp.float32)
                for cc in range(1, FW // 16):
                    feat_v[r, pl.ds(cc * 16, 16)] = z
                return carry
            lax.fori_loop(0, K * _CH, zrow, 0)

        def chunk_body(c, carry):
            gbase = pbase + c * _CH
            cp1 = pltpu.async_copy(
                xp_hbm.at[idx_v.at[pl.ds(c * (_CH * K), HK)]],
                rows_v.at[pl.ds(0, HK)], sem)
            cp2 = pltpu.async_copy(
                xp_hbm.at[idx_v.at[pl.ds(c * (_CH * K) + HK, HK)]],
                rows_v.at[pl.ds(HK, HK)], sem)
            pltpu.sync_copy(xp_hbm.at[pl.ds(gbase, _CH)], xc_v)
            if C < 16:
                pltpu.sync_copy(xsh_hbm.at[pl.ds(gbase, _CH)], xsh_v)
            cp1.wait()
            cp2.wait()

            if C < 16:
                iota = lax.iota(jnp.int32, 16)
                for p in range(_CH):
                    xc = xc_v[p, pl.ds(0, 16)]
                    sh = xsh_v[p, pl.ds(0, 16)]
                    for j in range(K):
                        r = rows_v[p * K + j, pl.ds(0, 16)]
                        feat_v[j * _CH + p, pl.ds(0, 16)] = jnp.where(
                            iota < C, r - xc, sh)
            else:
                def cc_body(cc, carry2):
                    off = cc * 16
                    for p in range(_CH):
                        xc = xc_v[p, pl.ds(off, 16)]
                        for j in range(K):
                            r = rows_v[p * K + j, pl.ds(off, 16)]
                            feat_v[j * _CH + p, pl.ds(off, 16)] = r - xc
                            feat_v[j * _CH + p, pl.ds(C + off, 16)] = xc
                    return carry2
                lax.fori_loop(0, C // 16, cc_body, 0)

            for j in range(K):
                pltpu.sync_copy(feat_v.at[pl.ds(j * _CH, _CH)],
                                feat_hbm.at[pl.ds(j * BN + gbase, _CH)])
            return carry

        lax.fori_loop(0, NCH, chunk_body, 0)

    return sc_kernel(xp2, xsh, idxf)


def _tc_conv_reduce(feat, Wp, Cout):
    KBN, FW = feat.shape
    BN = KBN // K
    RN = 512

    def body(feat_ref, w_ref, mx_ref, mn_ref, ps1_ref, ps2_ref, s_scr, s2_scr):
        i = pl.program_id(0)
        j = pl.program_id(1)
        y = lax.dot_general(feat_ref[...], w_ref[...], (((1,), (1,)), ((), ())),
                            preferred_element_type=jnp.float32)

        @pl.when(j == 0)
        def _():
            mx_ref[...] = y
            mn_ref[...] = y
            s_scr[...] = y
            s2_scr[...] = y * y

        @pl.when(j > 0)
        def _():
            mx_ref[...] = jnp.maximum(mx_ref[...], y)
            mn_ref[...] = jnp.minimum(mn_ref[...], y)
            s_scr[...] = s_scr[...] + y
            s2_scr[...] = s2_scr[...] + y * y

        @pl.when(j == K - 1)
        def _():
            p1 = jnp.sum(s_scr[...], axis=0, keepdims=True)
            p2 = jnp.sum(s2_scr[...], axis=0, keepdims=True)

            @pl.when(i == 0)
            def _():
                ps1_ref[...] = p1
                ps2_ref[...] = p2

            @pl.when(i > 0)
            def _():
                ps1_ref[...] = ps1_ref[...] + p1
                ps2_ref[...] = ps2_ref[...] + p2

    return pl.pallas_call(
        body,
        grid=(BN // RN, K),
        in_specs=[
            pl.BlockSpec((RN, FW), lambda i, j: (j * (BN // RN) + i, 0)),
            pl.BlockSpec((Cout, FW), lambda i, j: (0, 0)),
        ],
        out_specs=[
            pl.BlockSpec((RN, Cout), lambda i, j: (i, 0)),
            pl.BlockSpec((RN, Cout), lambda i, j: (i, 0)),
            pl.BlockSpec((1, Cout), lambda i, j: (0, 0)),
            pl.BlockSpec((1, Cout), lambda i, j: (0, 0)),
        ],
        out_shape=[
            jax.ShapeDtypeStruct((BN, Cout), jnp.float32),
            jax.ShapeDtypeStruct((BN, Cout), jnp.float32),
            jax.ShapeDtypeStruct((1, Cout), jnp.float32),
            jax.ShapeDtypeStruct((1, Cout), jnp.float32),
        ],
        scratch_shapes=[
            pltpu.VMEM((RN, Cout), jnp.float32),
            pltpu.VMEM((RN, Cout), jnp.float32),
        ],
    )(feat, Wp)


def _tc_finalize(mx2, mn2, ps1, ps2, g, bvec, count):
    BN, Cout = mx2.shape
    RC = 1024
    g2 = g.reshape(1, Cout)
    b2 = bvec.reshape(1, Cout)

    def body(mx_ref, mn_ref, ps1_ref, ps2_ref, g_ref, b_ref, o_ref):
        s1 = ps1_ref[...]
        s2 = ps2_ref[...]
        mean = s1 / count
        var = s2 / count - mean * mean
        scale = g_ref[...] * lax.rsqrt(var + EPS)
        bias = b_ref[...] - mean * scale
        M = jnp.where(scale >= 0, mx_ref[...], mn_ref[...])
        z = M * scale + bias
        o_ref[...] = jnp.where(z >= 0, z, 0.2 * z)

    return pl.pallas_call(
        body,
        grid=(BN // RC,),
        in_specs=[
            pl.BlockSpec((RC, Cout), lambda i: (i, 0)),
            pl.BlockSpec((RC, Cout), lambda i: (i, 0)),
            pl.BlockSpec((1, Cout), lambda i: (0, 0)),
            pl.BlockSpec((1, Cout), lambda i: (0, 0)),
            pl.BlockSpec((1, Cout), lambda i: (0, 0)),
            pl.BlockSpec((1, Cout), lambda i: (0, 0)),
        ],
        out_specs=pl.BlockSpec((RC, Cout), lambda i: (i, 0)),
        out_shape=jax.ShapeDtypeStruct((BN, Cout), jnp.float32),
    )(mx2, mn2, ps1, ps2, g2, b2)


def _block_fast(xt, W, g, bvec):
    B, N, Cin = xt.shape
    Cout = W.shape[0]
    C2 = 2 * Cin
    FW = max(C2, 128)
    BN = B * N
    xs = jnp.swapaxes(xt, 1, 2)
    idx = _tc_knn(xt, xs)
    idxf = idx.reshape(BN * K)
    x2 = xt.reshape(BN, Cin)
    xp2 = jnp.zeros((BN, _XP), jnp.float32).at[:, :Cin].set(x2)
    if Cin < 16:
        xsh = jnp.zeros((BN, _XP), jnp.float32).at[:, Cin:C2].set(x2)
    else:
        xsh = xp2
    feat = _sc_feat(xp2, xsh, idxf, Cin, FW)
    Wp = jnp.zeros((Cout, FW), jnp.float32).at[:, :C2].set(W)
    mx2, mn2, ps1, ps2 = _tc_conv_reduce(feat, Wp, Cout)
    out2 = _tc_finalize(mx2, mn2, ps1, ps2, g, bvec, float(BN * K))
    return out2.reshape(B, N, Cout)


def kernel(x, W1, g1, b1, W2, g2, b2, W3, g3, b3, W4, g4, b4):
    h = x
    h = _block_fast(h, W1, g1, b1)
    h = _block_fast(h, W2, g2, b2)
    h = _block_fast(h, W3, g3, b3)
    h = _block_fast(h, W4, g4, b4)
    return h


# speedup vs baseline: 8.6104x; 1.1422x over previous
import functools

import jax
import jax.numpy as jnp
from jax import lax
from jax.experimental import pallas as pl
from jax.experimental.pallas import tpu as pltpu
from jax.experimental.pallas import tpu_sc as plsc

K = 20
EPS = 1e-5
_NEG = -1e30
_NW = 32
_CH = 8
_XP = 128


def _tc_knn(xt, xs):
    B, N, Cin = xt.shape
    R = 256

    def body(xt_tile_ref, xs_full_ref, idx_ref):
        b = pl.program_id(0)
        xt_t = xt_tile_ref[0]
        xsf = xs_full_ref[0]
        dot = lax.dot_general(xt_t, xsf, (((1,), (0,)), ((), ())),
                              preferred_element_type=jnp.float32)
        xxl = jnp.sum(xsf * xsf, axis=0, keepdims=True)
        d = 2.0 * dot - xxl
        lane = lax.broadcasted_iota(jnp.int32, (R, N), 1)
        cols = []
        for _ in range(K):
            m = jnp.max(d, axis=1, keepdims=True)
            cand = jnp.where(d == m, lane, N)
            am = jnp.min(cand, axis=1, keepdims=True)
            cols.append(am)
            d = jnp.where(cand == am, _NEG, d)
        idx_ref[0] = jnp.concatenate(cols, axis=1) + b * N

    return pl.pallas_call(
        body,
        grid=(B, N // R),
        in_specs=[
            pl.BlockSpec((1, R, Cin), lambda b, i: (b, i, 0)),
            pl.BlockSpec((1, Cin, N), lambda b, i: (b, 0, 0)),
        ],
        out_specs=pl.BlockSpec((1, R, K), lambda b, i: (b, i, 0)),
        out_shape=jax.ShapeDtypeStruct((B, N, K), jnp.int32),
    )(xt, xs)


def _sc_gather(xp2, idxf):
    BN = xp2.shape[0]
    PW = BN // _NW
    NCH = PW // _CH
    HK = _CH * K // 2
    mesh = plsc.VectorSubcoreMesh(core_axis_name="c", subcore_axis_name="s")

    @functools.partial(
        pl.kernel, mesh=mesh,
        out_type=jax.ShapeDtypeStruct((BN * K, _XP), jnp.float32),
        scratch_types=[
            pltpu.VMEM((PW * K,), jnp.int32),
            pltpu.VMEM((_CH * K, _XP), jnp.float32),
            pltpu.SemaphoreType.DMA,
        ],
    )
    def sc_kernel(xp_hbm, idx_hbm, feat_hbm, idx_v, rows_v, sem):
        wid = lax.axis_index("s") * 2 + lax.axis_index("c")
        pbase = wid * PW
        pltpu.sync_copy(idx_hbm.at[pl.ds(pbase * K, PW * K)], idx_v)

        def chunk_body(c, carry):
            gbase = pbase + c * _CH
            cp1 = pltpu.async_copy(
                xp_hbm.at[idx_v.at[pl.ds(c * (_CH * K), HK)]],
                rows_v.at[pl.ds(0, HK)], sem)
            cp2 = pltpu.async_copy(
                xp_hbm.at[idx_v.at[pl.ds(c * (_CH * K) + HK, HK)]],
                rows_v.at[pl.ds(HK, HK)], sem)
            cp1.wait()
            cp2.wait()
            pltpu.sync_copy(rows_v,
                            feat_hbm.at[pl.ds(gbase * K, _CH * K)])
            return carry

        lax.fori_loop(0, NCH, chunk_body, 0)

    return sc_kernel(xp2, idxf)


def _tc_conv_reduce(feat, xp2, xsh, Wp, C, Cout, FW):
    KBN = feat.shape[0]
    BN = KBN // K
    PT = 32
    RN = PT * K

    def body(feat_ref, xc_ref, xsh_ref, w_ref, mx_ref, mn_ref, ps1_ref,
             ps2_ref):
        i = pl.program_id(0)
        f = feat_ref[...]
        xcb = jnp.broadcast_to(xc_ref[...][:, None, :],
                               (PT, K, _XP)).reshape(RN, _XP)
        if 2 * C <= _XP:
            shb = jnp.broadcast_to(xsh_ref[...][:, None, :],
                                   (PT, K, _XP)).reshape(RN, _XP)
            lane = lax.broadcasted_iota(jnp.int32, (RN, _XP), 1)
            e = jnp.where(lane < C, f - xcb, shb)
        else:
            e = jnp.concatenate([f - xcb, xcb], axis=1)
        y = lax.dot_general(e, w_ref[...], (((1,), (1,)), ((), ())),
                            preferred_element_type=jnp.float32)
        t = y.reshape(PT, K, Cout)
        mx_ref[...] = jnp.max(t, axis=1)
        mn_ref[...] = jnp.min(t, axis=1)
        p1 = jnp.sum(t, axis=(0, 1), keepdims=False).reshape(1, Cout)
        p2 = jnp.sum(t * t, axis=(0, 1), keepdims=False).reshape(1, Cout)

        @pl.when(i == 0)
        def _():
            ps1_ref[...] = p1
            ps2_ref[...] = p2

        @pl.when(i > 0)
        def _():
            ps1_ref[...] = ps1_ref[...] + p1
            ps2_ref[...] = ps2_ref[...] + p2

    return pl.pallas_call(
        body,
        grid=(BN // PT,),
        in_specs=[
            pl.BlockSpec((RN, _XP), lambda i: (i, 0)),
            pl.BlockSpec((PT, _XP), lambda i: (i, 0)),
            pl.BlockSpec((PT, _XP), lambda i: (i, 0)),
            pl.BlockSpec((Cout, FW), lambda i: (0, 0)),
        ],
        out_specs=[
            pl.BlockSpec((PT, Cout), lambda i: (i, 0)),
            pl.BlockSpec((PT, Cout), lambda i: (i, 0)),
            pl.BlockSpec((1, Cout), lambda i: (0, 0)),
            pl.BlockSpec((1, Cout), lambda i: (0, 0)),
        ],
        out_shape=[
            jax.ShapeDtypeStruct((BN, Cout), jnp.float32),
            jax.ShapeDtypeStruct((BN, Cout), jnp.float32),
            jax.ShapeDtypeStruct((1, Cout), jnp.float32),
            jax.ShapeDtypeStruct((1, Cout), jnp.float32),
        ],
    )(feat, xp2, xsh, Wp)


def _tc_finalize(mx2, mn2, ps1, ps2, g, bvec, count):
    BN, Cout = mx2.shape
    RC = 1024
    g2 = g.reshape(1, Cout)
    b2 = bvec.reshape(1, Cout)

    def body(mx_ref, mn_ref, ps1_ref, ps2_ref, g_ref, b_ref, o_ref):
        s1 = ps1_ref[...]
        s2 = ps2_ref[...]
        mean = s1 / count
        var = s2 / count - mean * mean
        scale = g_ref[...] * lax.rsqrt(var + EPS)
        bias = b_ref[...] - mean * scale
        M = jnp.where(scale >= 0, mx_ref[...], mn_ref[...])
        z = M * scale + bias
        o_ref[...] = jnp.where(z >= 0, z, 0.2 * z)

    return pl.pallas_call(
        body,
        grid=(BN // RC,),
        in_specs=[
            pl.BlockSpec((RC, Cout), lambda i: (i, 0)),
            pl.BlockSpec((RC, Cout), lambda i: (i, 0)),
            pl.BlockSpec((1, Cout), lambda i: (0, 0)),
            pl.BlockSpec((1, Cout), lambda i: (0, 0)),
            pl.BlockSpec((1, Cout), lambda i: (0, 0)),
            pl.BlockSpec((1, Cout), lambda i: (0, 0)),
        ],
        out_specs=pl.BlockSpec((RC, Cout), lambda i: (i, 0)),
        out_shape=jax.ShapeDtypeStruct((BN, Cout), jnp.float32),
    )(mx2, mn2, ps1, ps2, g2, b2)


def _block_fast(xt, W, g, bvec):
    B, N, Cin = xt.shape
    Cout = W.shape[0]
    BN = B * N
    xs = jnp.swapaxes(xt, 1, 2)
    idx = _tc_knn(xt, xs)
    idxf = idx.reshape(BN * K)
    x2 = xt.reshape(BN, Cin)
    C2 = 2 * Cin
    FW = max(C2, _XP)
    xp2 = jnp.zeros((BN, _XP), jnp.float32).at[:, :Cin].set(x2)
    if C2 <= _XP:
        xsh = jnp.zeros((BN, _XP), jnp.float32).at[:, Cin:C2].set(x2)
    else:
        xsh = xp2
    feat = _sc_gather(xp2, idxf)
    Wp = jnp.zeros((Cout, FW), jnp.float32).at[:, :C2].set(W)
    mx2, mn2, ps1, ps2 = _tc_conv_reduce(feat, xp2, xsh, Wp, Cin, Cout, FW)
    out2 = _tc_finalize(mx2, mn2, ps1, ps2, g, bvec, float(BN * K))
    return out2.reshape(B, N, Cout)


def kernel(x, W1, g1, b1, W2, g2, b2, W3, g3, b3, W4, g4, b4):
    h = x
    h = _block_fast(h, W1, g1, b1)
    h = _block_fast(h, W2, g2, b2)
    h = _block_fast(h, W3, g3, b3)
    h = _block_fast(h, W4, g4, b4)
    return h
